# baseline (speedup 1.0000x reference)
import functools

import jax
import jax.numpy as jnp
import numpy as np
from jax import lax
from jax.experimental import pallas as pl
from jax.experimental.pallas import tpu as pltpu
from jax.experimental.pallas import tpu_sc as plsc

_VOCAB = 1000000
_EMB = 64
_L = 50
_RAD = 5
_NEG = 5
_B = 4096

_NC = 2
_NS = 16
_NW = _NC * _NS
_POS_ROWS = _B * _L
_NEG_ROWS = 2 * _B * _NEG
_ROWS = _POS_ROWS + _NEG_ROWS
_RPW = _ROWS // _NW
_CH = 128
_NCH = _RPW // _CH

_T = _L // 2
_BSUB = 2048
_NB2 = _B // _BSUB
_NPROD = 11
_NCOL = 2 * _NPROD


def _sc_gather(table, idx3):
    mesh = plsc.VectorSubcoreMesh(core_axis_name="c", subcore_axis_name="s")
    hp = _CH // 2

    @functools.partial(
        pl.kernel,
        mesh=mesh,
        compiler_params=pltpu.CompilerParams(use_tc_tiling_on_sc=False),
        out_type=jax.ShapeDtypeStruct((_ROWS // 2, 2 * _EMB), jnp.float32),
        scratch_types=[
            pltpu.VMEM((_NCH, _CH), jnp.int32),
            pltpu.VMEM((_CH, _EMB), jnp.float32),
            pltpu.VMEM((_CH, _EMB), jnp.float32),
            pltpu.SemaphoreType.DMA,
            pltpu.SemaphoreType.DMA,
        ],
    )
    def gather_kernel(table_hbm, idx_hbm, out_hbm, idx_v, rows_a, rows_b,
                      sem_a, sem_b):
        wid = lax.axis_index("s") * _NC + lax.axis_index("c")
        pltpu.sync_copy(idx_hbm.at[wid], idx_v)
        pbase = wid * (_RPW // 2)

        def start(j, buf, sem):
            pltpu.async_copy(table_hbm.at[idx_v.at[j]], buf, sem)

        def finish(j, buf, sem):
            pltpu.make_async_copy(table_hbm.at[idx_v.at[j]], buf, sem).wait()
            pr = pbase + j * hp
            pltpu.sync_copy(buf.at[pl.ds(0, hp)],
                            out_hbm.at[pl.ds(pr, hp), pl.ds(0, _EMB)])
            pltpu.sync_copy(buf.at[pl.ds(hp, hp)],
                            out_hbm.at[pl.ds(pr, hp), pl.ds(_EMB, _EMB)])

        start(0, rows_a, sem_a)

        def body(m, carry):
            j0 = 2 * m
            start(j0 + 1, rows_b, sem_b)
            finish(j0, rows_a, sem_a)
            start(jnp.minimum(j0 + 2, _NCH - 1), rows_a, sem_a)
            finish(j0 + 1, rows_b, sem_b)
            return carry

        lax.fori_loop(0, _NCH // 2, body, 0)
        pltpu.make_async_copy(
            table_hbm.at[idx_v.at[_NCH - 1]], rows_a, sem_a).wait()

    return gather_kernel(table, idx3)


def _sel_matrix():
    sel = np.zeros((_NPROD * 128, 128), np.float32)
    for p in range(_NPROD):
        sel[p * 128: p * 128 + 64, 2 * p] = 1.0
        sel[p * 128 + 64: (p + 1) * 128, 2 * p + 1] = 1.0
    return jnp.asarray(sel)


_POS_TMAX = [23, 23, 22, 22, 24, -1, 23, 23, 22, 22, -1, 21]


def _tc_loss(g2):
    nblk = _ROWS // 2 // _BSUB
    negblk0 = _POS_ROWS // 2 // _BSUB

    def body(a_ref, b1_ref, b2_ref, b3_ref, n0, n1, n2, n3, n4, sel_ref,
             out_ref, s_ref):
        i2 = pl.program_id(0)
        t = pl.program_id(1)
        a = a_ref[...]
        prods = [
            a * b1_ref[...],
            a * b2_ref[...],
            a * pltpu.roll(a, 64, 1),
            a * pltpu.roll(b1_ref[...], 64, 1),
            a * pltpu.roll(b2_ref[...], 64, 1),
            a * pltpu.roll(b3_ref[...], 64, 1),
            a * n0[...],
            a * n1[...],
            a * n2[...],
            a * n3[...],
            a * n4[...],
        ]
        for p in range(_NPROD):
            s_ref[:, p * 128:(p + 1) * 128] = prods[p]
        sims = jnp.dot(s_ref[...], sel_ref[...],
                       preferred_element_type=jnp.float32)

        p_ = jax.nn.sigmoid(sims)
        f = jnp.where(p_ > 0, -jnp.log(jnp.where(p_ > 0, p_, 1.0)), 100.0)
        q_ = 1.0 - p_
        g = jnp.where(q_ > 0, -jnp.log(jnp.where(q_ > 0, q_, 1.0)), 100.0)

        lanes = lax.broadcasted_iota(jnp.int32, (1, 128), 1)
        tmax = jnp.full((1, 128), -1, jnp.int32)
        for c, tm in enumerate(_POS_TMAX):
            tmax = jnp.where(lanes == c, tm, tmax)
        is_pos = lanes < 12
        is_neg = (lanes >= 12) & (lanes < _NCOL)
        w = jnp.where(is_pos & (t <= tmax), 2.0,
                      jnp.where(is_neg, 1.0, 0.0))
        vals = jnp.where(is_pos, f, g) * w
        part = jnp.sum(vals, axis=0, keepdims=True)

        @pl.when((i2 == 0) & (t == 0))
        def _():
            out_ref[...] = jnp.zeros_like(out_ref)

        out_ref[...] += part

    bspec = lambda im: pl.BlockSpec((_BSUB, 128), im)
    out = pl.pallas_call(
        body,
        grid=(_NB2, _T),
        in_specs=[
            bspec(lambda i2, t: (t * _NB2 + i2, 0)),
            bspec(lambda i2, t: (jnp.minimum(t + 1, _T - 1) * _NB2 + i2, 0)),
            bspec(lambda i2, t: (jnp.minimum(t + 2, _T - 1) * _NB2 + i2, 0)),
            bspec(lambda i2, t: (jnp.minimum(t + 3, _T - 1) * _NB2 + i2, 0)),
            bspec(lambda i2, t: (negblk0 + 0 * _NB2 + i2, 0)),
            bspec(lambda i2, t: (negblk0 + 1 * _NB2 + i2, 0)),
            bspec(lambda i2, t: (negblk0 + 2 * _NB2 + i2, 0)),
            bspec(lambda i2, t: (negblk0 + 3 * _NB2 + i2, 0)),
            bspec(lambda i2, t: (negblk0 + 4 * _NB2 + i2, 0)),
            pl.BlockSpec((_NPROD * 128, 128), lambda i2, t: (0, 0)),
        ],
        out_specs=pl.BlockSpec((1, 128), lambda i2, t: (0, 0)),
        out_shape=jax.ShapeDtypeStruct((1, 128), jnp.float32),
        scratch_shapes=[pltpu.VMEM((_BSUB, _NPROD * 128), jnp.float32)],
    )(g2, g2, g2, g2, g2, g2, g2, g2, g2, _sel_matrix())
    pos_sum = jnp.sum(out[0, :12])
    neg_sum = jnp.sum(out[0, 12:_NCOL])
    return pos_sum / (_B * _L * _L) + neg_sum / (_B * _L * _NEG)


def kernel(batch, table):
    neg_words = jax.random.randint(
        jax.random.key(1), (_B, _NEG), 1, _VOCAB, dtype=jnp.int32)
    pos_idx = (batch.T.reshape(_T, 2, _B)
               .transpose(0, 2, 1).reshape(-1))
    neg_idx = jnp.broadcast_to(
        neg_words.T.reshape(_NEG, _B, 1), (_NEG, _B, 2)).reshape(-1)
    idx = jnp.concatenate([pos_idx, neg_idx])
    idx3 = (idx.reshape(_NW, _NCH, _CH // 2, 2)
            .transpose(0, 1, 3, 2).reshape(_NW, _NCH, _CH))
    g2 = _sc_gather(table, idx3)
    return _tc_loss(g2)
